# baseline (speedup 1.0000x reference)
import functools

import jax
import jax.numpy as jnp
from jax import lax
from jax.experimental import pallas as pl
from jax.experimental.pallas import tpu as pltpu
from jax.experimental.pallas import tpu_sc as plsc

NUM_CORES = 2
NUM_SUBCORES = 16
NUM_WORKERS = NUM_CORES * NUM_SUBCORES
ROWS_PER_GROUP = 2
SPLITS = ((0, 128), (128, 72))
K = ROWS_PER_GROUP * len(SPLITS)


def _embed_kernel(rows_per_worker, row_len, d,
                  table_hbm, x_hbm, out_hbm,
                  idx_v, a0, a1, a2, a3, b0, b1, b2, b3,
                  gsem_a, gsem_b, wsem_a, wsem_b):
    cid = lax.axis_index("c")
    sid = lax.axis_index("s")
    wid = sid * NUM_CORES + cid
    row0 = wid * rows_per_worker
    bufs_a = (a0, a1, a2, a3)
    bufs_b = (b0, b1, b2, b3)
    n_groups = rows_per_worker // ROWS_PER_GROUP

    pltpu.sync_copy(x_hbm.at[pl.ds(row0, rows_per_worker)], idx_v)

    def chunk_refs(g, j, bufs):
        r = g * ROWS_PER_GROUP + j // len(SPLITS)
        off, sz = SPLITS[j % len(SPLITS)]
        src = table_hbm.at[idx_v.at[r, pl.ds(off, sz)]]
        dst = out_hbm.at[row0 + r, pl.ds(off, sz)]
        return src, bufs[j], dst

    def fire_g(g, bufs, sem):
        for j in range(K):
            src, buf, _ = chunk_refs(g, j, bufs)
            pltpu.async_copy(src, buf, sem)

    def drain_g(g, bufs, sem):
        for j in range(K):
            src, buf, _ = chunk_refs(g, j, bufs)
            pltpu.make_async_copy(src, buf, sem).wait()

    def fire_w(g, bufs, sem):
        for j in range(K):
            _, buf, dst = chunk_refs(g, j, bufs)
            pltpu.async_copy(buf, dst, sem)

    def drain_w(g, bufs, sem):
        for j in range(K):
            _, buf, dst = chunk_refs(g, j, bufs)
            pltpu.make_async_copy(buf, dst, sem).wait()

    fire_g(0, bufs_a, gsem_a)
    drain_g(0, bufs_a, gsem_a)
    fire_w(0, bufs_a, wsem_a)
    fire_g(1, bufs_b, gsem_b)

    def body(i, carry):
        g0 = 2 * i + 1
        g1 = g0 + 1
        drain_g(g0, bufs_b, gsem_b)
        fire_w(g0, bufs_b, wsem_b)
        drain_w(g0 - 1, bufs_a, wsem_a)
        fire_g(g1, bufs_a, gsem_a)
        drain_g(g1, bufs_a, gsem_a)
        fire_w(g1, bufs_a, wsem_a)
        drain_w(g1 - 1, bufs_b, wsem_b)
        fire_g(g1 + 1, bufs_b, gsem_b)
        return carry

    lax.fori_loop(0, (n_groups - 2) // 2, body, 0)

    g_last = n_groups - 1
    drain_g(g_last, bufs_b, gsem_b)
    fire_w(g_last, bufs_b, wsem_b)
    drain_w(g_last - 1, bufs_a, wsem_a)
    drain_w(g_last, bufs_b, wsem_b)


def kernel(x, table):
    n_rows, row_len = x.shape
    d = table.shape[1]
    xi = x.astype(jnp.int32)
    rows_per_worker = n_rows // NUM_WORKERS
    assert n_rows == NUM_WORKERS * rows_per_worker
    assert rows_per_worker % (2 * ROWS_PER_GROUP) == 0
    assert sum(sz for _, sz in SPLITS) == row_len

    mesh = plsc.VectorSubcoreMesh(core_axis_name="c", subcore_axis_name="s")
    buf_types = [pltpu.VMEM((sz, d), jnp.float32)
                 for _ in range(ROWS_PER_GROUP) for _, sz in SPLITS]
    out = pl.kernel(
        functools.partial(_embed_kernel, rows_per_worker, row_len, d),
        out_type=jax.ShapeDtypeStruct((n_rows, row_len, d), jnp.float32),
        mesh=mesh,
        scratch_types=[pltpu.VMEM((rows_per_worker, row_len), jnp.int32)]
        + buf_types + buf_types
        + [pltpu.SemaphoreType.DMA] * 4,
        compiler_params=pltpu.CompilerParams(use_tc_tiling_on_sc=False),
    )(table, xi)
    return out


# speedup vs baseline: 1.2257x; 1.2257x over previous
import functools

import jax
import jax.numpy as jnp
from jax import lax
from jax.experimental import pallas as pl
from jax.experimental.pallas import tpu as pltpu
from jax.experimental.pallas import tpu_sc as plsc

NUM_CORES = 2
NUM_SUBCORES = 16
NUM_WORKERS = NUM_CORES * NUM_SUBCORES
ROWS_PER_GROUP = 2
SPLITS = ((0, 128), (128, 72))
K = ROWS_PER_GROUP * len(SPLITS)
DPAD = 128


def _embed_kernel(rows_per_worker, row_len,
                  table_hbm, x_hbm, out_hbm,
                  idx_v, a0, a1, a2, a3, b0, b1, b2, b3,
                  gsem_a, gsem_b, wsem_a, wsem_b):
    cid = lax.axis_index("c")
    sid = lax.axis_index("s")
    wid = sid * NUM_CORES + cid
    row0 = wid * rows_per_worker
    flat0 = row0 * row_len
    bufs_a = (a0, a1, a2, a3)
    bufs_b = (b0, b1, b2, b3)
    n_groups = rows_per_worker // ROWS_PER_GROUP

    pltpu.sync_copy(x_hbm.at[pl.ds(row0, rows_per_worker)], idx_v)

    def chunk_refs(g, j, bufs):
        r = g * ROWS_PER_GROUP + j // len(SPLITS)
        off, sz = SPLITS[j % len(SPLITS)]
        src = table_hbm.at[idx_v.at[r, pl.ds(off, sz)]]
        dst = out_hbm.at[pl.ds(flat0 + r * row_len + off, sz)]
        return src, bufs[j], dst

    def fire_g(g, bufs, sem):
        for j in range(K):
            src, buf, _ = chunk_refs(g, j, bufs)
            pltpu.async_copy(src, buf, sem)

    def drain_g(g, bufs, sem):
        for j in range(K):
            src, buf, _ = chunk_refs(g, j, bufs)
            pltpu.make_async_copy(src, buf, sem).wait()

    def fire_w(g, bufs, sem):
        for j in range(K):
            _, buf, dst = chunk_refs(g, j, bufs)
            pltpu.async_copy(buf, dst, sem)

    def drain_w(g, bufs, sem):
        for j in range(K):
            _, buf, dst = chunk_refs(g, j, bufs)
            pltpu.make_async_copy(buf, dst, sem).wait()

    fire_g(0, bufs_a, gsem_a)
    drain_g(0, bufs_a, gsem_a)
    fire_w(0, bufs_a, wsem_a)
    fire_g(1, bufs_b, gsem_b)

    def body(i, carry):
        g0 = 2 * i + 1
        g1 = g0 + 1
        drain_g(g0, bufs_b, gsem_b)
        fire_w(g0, bufs_b, wsem_b)
        drain_w(g0 - 1, bufs_a, wsem_a)
        fire_g(g1, bufs_a, gsem_a)
        drain_g(g1, bufs_a, gsem_a)
        fire_w(g1, bufs_a, wsem_a)
        drain_w(g1 - 1, bufs_b, wsem_b)
        fire_g(g1 + 1, bufs_b, gsem_b)
        return carry

    lax.fori_loop(0, (n_groups - 2) // 2, body, 0)

    g_last = n_groups - 1
    drain_g(g_last, bufs_b, gsem_b)
    fire_w(g_last, bufs_b, wsem_b)
    drain_w(g_last - 1, bufs_a, wsem_a)
    drain_w(g_last, bufs_b, wsem_b)


def kernel(x, table):
    n_rows, row_len = x.shape
    d = table.shape[1]
    xi = x.astype(jnp.int32)
    tpad = jnp.pad(table, ((0, 0), (0, DPAD - d)))
    rows_per_worker = n_rows // NUM_WORKERS
    b_total = n_rows * row_len
    assert n_rows == NUM_WORKERS * rows_per_worker
    assert rows_per_worker % (2 * ROWS_PER_GROUP) == 0
    assert sum(sz for _, sz in SPLITS) == row_len

    mesh = plsc.VectorSubcoreMesh(core_axis_name="c", subcore_axis_name="s")
    buf_types = [pltpu.VMEM((sz, DPAD), jnp.float32)
                 for _ in range(ROWS_PER_GROUP) for _, sz in SPLITS]
    out = pl.kernel(
        functools.partial(_embed_kernel, rows_per_worker, row_len),
        out_type=jax.ShapeDtypeStruct((b_total, DPAD), jnp.float32),
        mesh=mesh,
        scratch_types=[pltpu.VMEM((rows_per_worker, row_len), jnp.int32)]
        + buf_types + buf_types
        + [pltpu.SemaphoreType.DMA] * 4,
        compiler_params=pltpu.CompilerParams(use_tc_tiling_on_sc=False),
    )(tpad, xi)
    return out.reshape(n_rows, row_len, DPAD)[:, :, :d]


# speedup vs baseline: 1.8126x; 1.4788x over previous
import functools

import jax
import jax.numpy as jnp
from jax import lax
from jax.experimental import layout as jex_layout
from jax.experimental import pallas as pl
from jax.experimental.pallas import tpu as pltpu
from jax.experimental.pallas import tpu_sc as plsc

NUM_CORES = 2
NUM_SUBCORES = 16
NUM_WORKERS = NUM_CORES * NUM_SUBCORES
ROWS_PER_GROUP = 2
SPLITS = ((0, 128), (128, 72))
K = ROWS_PER_GROUP * len(SPLITS)
DPAD = 128


def _embed_kernel(rows_per_worker, row_len, d,
                  table_hbm, x_hbm, out_hbm,
                  idx_v, a0, a1, a2, a3, b0, b1, b2, b3,
                  gsem_a, gsem_b, wsem_a, wsem_b):
    cid = lax.axis_index("c")
    sid = lax.axis_index("s")
    wid = sid * NUM_CORES + cid
    row0 = wid * rows_per_worker
    flat0 = row0 * row_len
    bufs_a = (a0, a1, a2, a3)
    bufs_b = (b0, b1, b2, b3)
    n_groups = rows_per_worker // ROWS_PER_GROUP

    pltpu.sync_copy(x_hbm.at[pl.ds(row0, rows_per_worker)], idx_v)

    def chunk_refs(g, j, bufs):
        r = g * ROWS_PER_GROUP + j // len(SPLITS)
        off, sz = SPLITS[j % len(SPLITS)]
        src = table_hbm.at[idx_v.at[r, pl.ds(off, sz)]]
        dst = out_hbm.at[pl.ds(flat0 + r * row_len + off, sz), pl.ds(0, d)]
        return src, bufs[j], dst

    def fire_g(g, bufs, sem):
        for j in range(K):
            src, buf, _ = chunk_refs(g, j, bufs)
            pltpu.async_copy(src, buf, sem)

    def drain_g(g, bufs, sem):
        for j in range(K):
            src, buf, _ = chunk_refs(g, j, bufs)
            pltpu.make_async_copy(src, buf, sem).wait()

    def fire_w(g, bufs, sem):
        for j in range(K):
            _, buf, dst = chunk_refs(g, j, bufs)
            pltpu.async_copy(buf, dst, sem)

    def drain_w(g, bufs, sem):
        for j in range(K):
            _, buf, dst = chunk_refs(g, j, bufs)
            pltpu.make_async_copy(buf, dst, sem).wait()

    fire_g(0, bufs_a, gsem_a)
    drain_g(0, bufs_a, gsem_a)
    fire_w(0, bufs_a, wsem_a)
    fire_g(1, bufs_b, gsem_b)

    def body(i, carry):
        g0 = 2 * i + 1
        g1 = g0 + 1
        drain_g(g0, bufs_b, gsem_b)
        fire_w(g0, bufs_b, wsem_b)
        drain_w(g0 - 1, bufs_a, wsem_a)
        fire_g(g1, bufs_a, gsem_a)
        drain_g(g1, bufs_a, gsem_a)
        fire_w(g1, bufs_a, wsem_a)
        drain_w(g1 - 1, bufs_b, wsem_b)
        fire_g(g1 + 1, bufs_b, gsem_b)
        return carry

    lax.fori_loop(0, (n_groups - 2) // 2, body, 0)

    g_last = n_groups - 1
    drain_g(g_last, bufs_b, gsem_b)
    fire_w(g_last, bufs_b, wsem_b)
    drain_w(g_last - 1, bufs_a, wsem_a)
    drain_w(g_last, bufs_b, wsem_b)


def kernel(x, table):
    n_rows, row_len = x.shape
    d = table.shape[1]
    xi = x.astype(jnp.int32)
    lin = jex_layout.Layout(major_to_minor=(0, 1), tiling=())
    tlin = jex_layout.with_layout_constraint(table, lin)
    rows_per_worker = n_rows // NUM_WORKERS
    b_total = n_rows * row_len
    assert n_rows == NUM_WORKERS * rows_per_worker
    assert rows_per_worker % (2 * ROWS_PER_GROUP) == 0
    assert sum(sz for _, sz in SPLITS) == row_len

    mesh = plsc.VectorSubcoreMesh(core_axis_name="c", subcore_axis_name="s")
    buf_types = [pltpu.VMEM((sz, d), jnp.float32)
                 for _ in range(ROWS_PER_GROUP) for _, sz in SPLITS]
    out = pl.kernel(
        functools.partial(_embed_kernel, rows_per_worker, row_len, d),
        out_type=jax.ShapeDtypeStruct((b_total, DPAD), jnp.float32),
        mesh=mesh,
        scratch_types=[pltpu.VMEM((rows_per_worker, row_len), jnp.int32)]
        + buf_types + buf_types
        + [pltpu.SemaphoreType.DMA] * 4,
        compiler_params=pltpu.CompilerParams(use_tc_tiling_on_sc=False),
    )(tlin, xi)
    return out.reshape(n_rows, row_len, DPAD)[:, :, :d]
